# Initial kernel scaffold; baseline (speedup 1.0000x reference)
import jax
import jax.numpy as jnp
from jax.experimental import pallas as pl


def kernel(x, edge_index, W1, a_src1, a_dst1, b1, W2, a_src2, a_dst2, b2, lw1, lb1, lw2, lb2):
    raise NotImplementedError("write your pallas kernel here")


# speedup vs baseline: 40.0946x; 40.0946x over previous
import functools

import jax
import jax.numpy as jnp
from jax import lax
from jax.experimental import pallas as pl
from jax.experimental.pallas import tpu as pltpu
from jax.experimental.pallas import tpu_sc as plsc

N = 10000
E = 320000
C = 128
H = 2
NS = 16
EPT = E // NS
K = 80
NB = EPT // K
NPT = N // NS
R = 144
F16 = C // 16
SEGB = 10
NSEG = NB // SEGB


def _node_strips():
    out = []
    off = 0
    while off < NPT:
        L = min(K, NPT - off)
        out.append((off, L))
        off += L
    return out


def _sc_gat_layer(h, atab, src_r, dst_r, b2):
    mesh = plsc.VectorSubcoreMesh(core_axis_name="c", subcore_axis_name="s",
                                  num_cores=2, num_subcores=NS)

    @functools.partial(
        pl.kernel,
        out_type=(jax.ShapeDtypeStruct((2 * N, C), jnp.float32),
                  jax.ShapeDtypeStruct((H, NS, NB, K), jnp.float32)),
        mesh=mesh,
        compiler_params=pltpu.CompilerParams(use_tc_tiling_on_sc=False,
                                             needs_layout_passes=False),
        scratch_types=[
            pltpu.VMEM_SHARED((N, R), jnp.float32),
            pltpu.VMEM((SEGB, K), jnp.int32),
            pltpu.VMEM((SEGB, K), jnp.int32),
            pltpu.VMEM((SEGB, K), jnp.float32),
            pltpu.VMEM((H, C), jnp.float32),
        ],
    )
    def k(h_hbm, atab_hbm, src_hbm, dst_hbm, b_hbm, out_hbm, ex_hbm,
          acc_s, srcs, dsts, exs, bv):
        c = lax.axis_index("c")
        s = lax.axis_index("s")
        n0 = s * NPT
        lane0 = lax.iota(jnp.int32, 16) == 0
        zf16 = jnp.zeros((16,), jnp.float32)
        cN = c * N

        pltpu.sync_copy(b_hbm, bv)

        def phase_a(atabv):
            pltpu.sync_copy(atab_hbm.at[c], atabv)

            @pl.loop(0, NSEG)
            def _(seg):
                pltpu.sync_copy(src_hbm.at[s].at[pl.ds(seg * SEGB, SEGB)],
                                srcs)
                pltpu.sync_copy(dst_hbm.at[s].at[pl.ds(seg * SEGB, SEGB)],
                                dsts)

                @pl.loop(0, SEGB)
                def _(bi):
                    for col in range(0, K, 16):
                        sv = srcs[bi, pl.ds(col, 16)]
                        dv = dsts[bi, pl.ds(col, 16)]
                        av = (plsc.load_gather(atabv, [sv])
                              + plsc.load_gather(atabv, [dv + N]))
                        av = jnp.where(av >= 0.0, av, 0.2 * av)
                        exs[bi, pl.ds(col, 16)] = jnp.exp(av)

                pltpu.sync_copy(
                    exs, ex_hbm.at[c].at[s].at[pl.ds(seg * SEGB, SEGB)])

        pl.run_scoped(phase_a, pltpu.VMEM((2 * N,), jnp.float32))

        def phase_b(rowsv, stagv):
            @pl.loop(0, K)
            def _(e):
                for j in range(R // 16):
                    stagv[e, pl.ds(16 * j, 16)] = zf16

            for off, L in _node_strips():
                pltpu.sync_copy(stagv.at[pl.ds(0, L)],
                                acc_s.at[pl.ds(n0 + off, L)])

            plsc.subcore_barrier()

            @pl.loop(0, NSEG)
            def _(seg):
                pltpu.sync_copy(src_hbm.at[s].at[pl.ds(seg * SEGB, SEGB)],
                                srcs)
                pltpu.sync_copy(dst_hbm.at[s].at[pl.ds(seg * SEGB, SEGB)],
                                dsts)
                pltpu.sync_copy(
                    ex_hbm.at[c].at[s].at[pl.ds(seg * SEGB, SEGB)], exs)

                @pl.loop(0, SEGB)
                def _(bi):
                    for col in range(0, K, 16):
                        srcs[bi, pl.ds(col, 16)] = (
                            srcs[bi, pl.ds(col, 16)] + cN)

                @pl.loop(0, SEGB)
                def _(bi):
                    pltpu.sync_copy(h_hbm.at[srcs.at[bi]], rowsv)

                    @pl.loop(0, K // 16)
                    def _(g):
                        ex16 = exs[bi, pl.ds(g * 16, 16)]
                        for r in range(16):
                            e = g * 16 + r
                            exb = jnp.broadcast_to(ex16[r], (16,))
                            for j in range(F16):
                                stagv[e, pl.ds(16 * j, 16)] = (
                                    rowsv[e, pl.ds(16 * j, 16)] * exb)
                            stagv[e, pl.ds(C, 16)] = jnp.where(
                                lane0, exb, 0.0)

                    pltpu.sync_copy(stagv, acc_s.at[dsts.at[bi]], add=True)

            plsc.subcore_barrier()

            b_vecs = [bv[c, pl.ds(16 * j, 16)] for j in range(F16)]
            for off, L in _node_strips():
                pltpu.sync_copy(acc_s.at[pl.ds(n0 + off, L)],
                                stagv.at[pl.ds(0, L)])

                @pl.loop(0, L)
                def _(i):
                    dv16 = stagv[i, pl.ds(C, 16)]
                    inv = 1.0 / (jnp.broadcast_to(dv16[0], (16,)) + 1e-16)
                    for j in range(F16):
                        rowsv[i, pl.ds(16 * j, 16)] = jnp.maximum(
                            stagv[i, pl.ds(16 * j, 16)] * inv + b_vecs[j],
                            0.0)

                pltpu.sync_copy(rowsv.at[pl.ds(0, L)],
                                out_hbm.at[pl.ds(c * N + n0 + off, L)])

        pl.run_scoped(phase_b,
                      pltpu.VMEM((K, C), jnp.float32),
                      pltpu.VMEM((K, R), jnp.float32))

    return k(h, atab, src_r, dst_r, b2)[0]


def _tc_mm(parts, A):
    def body(*refs):
        o_ref, o2_ref = refs[-2], refs[-1]
        a_ref = refs[-3]
        xw = None
        for i in range(len(parts)):
            p = jnp.dot(refs[2 * i][...], refs[2 * i + 1][...],
                        preferred_element_type=jnp.float32)
            xw = p if xw is None else xw + p
        o_ref[0:N, :] = xw[:, 0:C]
        o_ref[N:2 * N, :] = xw[:, C:2 * C]
        o2_ref[...] = jnp.dot(xw, a_ref[...],
                              preferred_element_type=jnp.float32)

    args = []
    for xi, wi in parts:
        args.extend([xi, wi])
    args.append(A)
    return pl.pallas_call(
        body,
        out_shape=(jax.ShapeDtypeStruct((2 * N, C), jnp.float32),
                   jax.ShapeDtypeStruct((N, 4), jnp.float32)))(*args)


def _tc_post(hin, lw1, lb1, lw2, lb2):
    def body(h_ref, w1_ref, b1_ref, w2_ref, b2_ref, o_ref):
        t = (jnp.dot(h_ref[0:N, :], w1_ref[0:C, :],
                     preferred_element_type=jnp.float32)
             + jnp.dot(h_ref[N:2 * N, :], w1_ref[C:2 * C, :],
                       preferred_element_type=jnp.float32))
        t = t + b1_ref[...][None, :]
        y = jnp.dot(t, w2_ref[...],
                    preferred_element_type=jnp.float32) + b2_ref[...][None, :]
        o_ref[...] = jax.nn.sigmoid(y)

    out_dim = lw2.shape[1]
    return pl.pallas_call(
        body, out_shape=jax.ShapeDtypeStruct((N, out_dim), jnp.float32))(
            hin, lw1, lb1, lw2, lb2)


def _pack_a(a_src, a_dst):
    z = jnp.zeros((C,), jnp.float32)
    cols = [jnp.concatenate([a_src[0], z]),
            jnp.concatenate([a_dst[0], z]),
            jnp.concatenate([z, a_src[1]]),
            jnp.concatenate([z, a_dst[1]])]
    return jnp.stack(cols, axis=1)


@jax.jit
def kernel(x, edge_index, W1, a_src1, a_dst1, b1,
           W2, a_src2, a_dst2, b2, lw1, lb1, lw2, lb2):
    src_r = edge_index[0].reshape(NS, NB, K)
    dst_r = edge_index[1].reshape(NS, NB, K)
    h1, atab1 = _tc_mm([(x, W1)], _pack_a(a_src1, a_dst1))
    g1 = _sc_gat_layer(h1, atab1.T.reshape(H, 2 * N), src_r, dst_r,
                       b1.reshape(H, C))
    h2, atab2 = _tc_mm([(g1[0:N], W2[0:C]), (g1[N:2 * N], W2[C:2 * C])],
                       _pack_a(a_src2, a_dst2))
    g2 = _sc_gat_layer(h2, atab2.T.reshape(H, 2 * N), src_r, dst_r,
                       b2.reshape(H, C))
    return _tc_post(g2, lw1, lb1, lw2, lb2)


# speedup vs baseline: 58.0845x; 1.4487x over previous
import functools

import jax
import jax.numpy as jnp
from jax import lax
from jax.experimental import pallas as pl
from jax.experimental.pallas import tpu as pltpu
from jax.experimental.pallas import tpu_sc as plsc

N = 10000
E = 320000
C = 128
H = 2
NS = 16
EPT = E // NS
K = 80
NB = EPT // K
NPT = N // NS
R = 144
F16 = C // 16
SEGB = 10
NSEG = NB // SEGB


def _node_strips():
    out = []
    off = 0
    while off < NPT:
        L = min(K, NPT - off)
        out.append((off, L))
        off += L
    return out


def _sc_gat_layer(h, atab, src_r, dst_r, b2):
    mesh = plsc.VectorSubcoreMesh(core_axis_name="c", subcore_axis_name="s",
                                  num_cores=2, num_subcores=NS)

    @functools.partial(
        pl.kernel,
        out_type=(jax.ShapeDtypeStruct((2 * N, C), jnp.float32),
                  jax.ShapeDtypeStruct((H, NS, NB, K), jnp.float32)),
        mesh=mesh,
        compiler_params=pltpu.CompilerParams(use_tc_tiling_on_sc=False,
                                             needs_layout_passes=False),
        scratch_types=[
            pltpu.VMEM_SHARED((N, R), jnp.float32),
            pltpu.VMEM((SEGB, K), jnp.int32),
            pltpu.VMEM((SEGB, K), jnp.int32),
            pltpu.VMEM((SEGB, K), jnp.float32),
            pltpu.VMEM((H, C), jnp.float32),
        ],
    )
    def k(h_hbm, atab_hbm, src_hbm, dst_hbm, b_hbm, out_hbm, ex_hbm,
          acc_s, srcs, dsts, exs, bv):
        c = lax.axis_index("c")
        s = lax.axis_index("s")
        n0 = s * NPT
        lane0 = lax.iota(jnp.int32, 16) == 0
        zf16 = jnp.zeros((16,), jnp.float32)
        cN = c * N

        pltpu.sync_copy(b_hbm, bv)

        def phase_a(atabv):
            pltpu.sync_copy(atab_hbm.at[c], atabv)

            @pl.loop(0, NSEG)
            def _(seg):
                pltpu.sync_copy(src_hbm.at[s].at[pl.ds(seg * SEGB, SEGB)],
                                srcs)
                pltpu.sync_copy(dst_hbm.at[s].at[pl.ds(seg * SEGB, SEGB)],
                                dsts)

                @pl.loop(0, SEGB)
                def _(bi):
                    for col in range(0, K, 16):
                        sv = srcs[bi, pl.ds(col, 16)]
                        dv = dsts[bi, pl.ds(col, 16)]
                        av = (plsc.load_gather(atabv, [sv])
                              + plsc.load_gather(atabv, [dv + N]))
                        av = jnp.where(av >= 0.0, av, 0.2 * av)
                        exs[bi, pl.ds(col, 16)] = jnp.exp(av)

                pltpu.sync_copy(
                    exs, ex_hbm.at[c].at[s].at[pl.ds(seg * SEGB, SEGB)])

        pl.run_scoped(phase_a, pltpu.VMEM((2 * N,), jnp.float32))

        def phase_b(rows0, rows1, stagv, sem0, sem1):
            rows = (rows0, rows1)
            sems = (sem0, sem1)
            rowsv = rows0
            @pl.loop(0, K)
            def _(e):
                for j in range(R // 16):
                    stagv[e, pl.ds(16 * j, 16)] = zf16

            for off, L in _node_strips():
                pltpu.sync_copy(stagv.at[pl.ds(0, L)],
                                acc_s.at[pl.ds(n0 + off, L)])

            plsc.subcore_barrier()

            @pl.loop(0, NSEG)
            def _(seg):
                pltpu.sync_copy(src_hbm.at[s].at[pl.ds(seg * SEGB, SEGB)],
                                srcs)
                pltpu.sync_copy(dst_hbm.at[s].at[pl.ds(seg * SEGB, SEGB)],
                                dsts)
                pltpu.sync_copy(
                    ex_hbm.at[c].at[s].at[pl.ds(seg * SEGB, SEGB)], exs)

                @pl.loop(0, SEGB)
                def _(bi):
                    for col in range(0, K, 16):
                        srcs[bi, pl.ds(col, 16)] = (
                            srcs[bi, pl.ds(col, 16)] + cN)

                descs = [None] * SEGB
                descs[0] = pltpu.async_copy(h_hbm.at[srcs.at[0]], rows[0],
                                            sems[0])
                for bi in range(SEGB):
                    p = bi & 1
                    descs[bi].wait()
                    if bi + 1 < SEGB:
                        descs[bi + 1] = pltpu.async_copy(
                            h_hbm.at[srcs.at[bi + 1]], rows[1 - p],
                            sems[1 - p])
                    rv = rows[p]

                    @pl.loop(0, K // 16)
                    def _(g):
                        ex16 = exs[bi, pl.ds(g * 16, 16)]
                        for r in range(16):
                            e = g * 16 + r
                            exb = jnp.broadcast_to(ex16[r], (16,))
                            for j in range(F16):
                                stagv[e, pl.ds(16 * j, 16)] = (
                                    rv[e, pl.ds(16 * j, 16)] * exb)
                            stagv[e, pl.ds(C, 16)] = jnp.where(
                                lane0, exb, 0.0)

                    pltpu.sync_copy(stagv, acc_s.at[dsts.at[bi]], add=True)

            plsc.subcore_barrier()

            b_vecs = [bv[c, pl.ds(16 * j, 16)] for j in range(F16)]
            for off, L in _node_strips():
                pltpu.sync_copy(acc_s.at[pl.ds(n0 + off, L)],
                                stagv.at[pl.ds(0, L)])

                @pl.loop(0, L)
                def _(i):
                    dv16 = stagv[i, pl.ds(C, 16)]
                    inv = 1.0 / (jnp.broadcast_to(dv16[0], (16,)) + 1e-16)
                    for j in range(F16):
                        rowsv[i, pl.ds(16 * j, 16)] = jnp.maximum(
                            stagv[i, pl.ds(16 * j, 16)] * inv + b_vecs[j],
                            0.0)

                pltpu.sync_copy(rowsv.at[pl.ds(0, L)],
                                out_hbm.at[pl.ds(c * N + n0 + off, L)])

        pl.run_scoped(phase_b,
                      pltpu.VMEM((K, C), jnp.float32),
                      pltpu.VMEM((K, C), jnp.float32),
                      pltpu.VMEM((K, R), jnp.float32),
                      pltpu.SemaphoreType.DMA,
                      pltpu.SemaphoreType.DMA)

    return k(h, atab, src_r, dst_r, b2)[0]


def _tc_mm(parts, A):
    def body(*refs):
        o_ref, o2_ref = refs[-2], refs[-1]
        a_ref = refs[-3]
        xw = None
        for i in range(len(parts)):
            p = jnp.dot(refs[2 * i][...], refs[2 * i + 1][...],
                        preferred_element_type=jnp.float32)
            xw = p if xw is None else xw + p
        o_ref[0:N, :] = xw[:, 0:C]
        o_ref[N:2 * N, :] = xw[:, C:2 * C]
        o2_ref[...] = jnp.dot(xw, a_ref[...],
                              preferred_element_type=jnp.float32)

    args = []
    for xi, wi in parts:
        args.extend([xi, wi])
    args.append(A)
    return pl.pallas_call(
        body,
        out_shape=(jax.ShapeDtypeStruct((2 * N, C), jnp.float32),
                   jax.ShapeDtypeStruct((N, 4), jnp.float32)))(*args)


def _tc_post(hin, lw1, lb1, lw2, lb2):
    def body(h_ref, w1_ref, b1_ref, w2_ref, b2_ref, o_ref):
        t = (jnp.dot(h_ref[0:N, :], w1_ref[0:C, :],
                     preferred_element_type=jnp.float32)
             + jnp.dot(h_ref[N:2 * N, :], w1_ref[C:2 * C, :],
                       preferred_element_type=jnp.float32))
        t = t + b1_ref[...][None, :]
        y = jnp.dot(t, w2_ref[...],
                    preferred_element_type=jnp.float32) + b2_ref[...][None, :]
        o_ref[...] = jax.nn.sigmoid(y)

    out_dim = lw2.shape[1]
    return pl.pallas_call(
        body, out_shape=jax.ShapeDtypeStruct((N, out_dim), jnp.float32))(
            hin, lw1, lb1, lw2, lb2)


def _pack_a(a_src, a_dst):
    z = jnp.zeros((C,), jnp.float32)
    cols = [jnp.concatenate([a_src[0], z]),
            jnp.concatenate([a_dst[0], z]),
            jnp.concatenate([z, a_src[1]]),
            jnp.concatenate([z, a_dst[1]])]
    return jnp.stack(cols, axis=1)


@jax.jit
def kernel(x, edge_index, W1, a_src1, a_dst1, b1,
           W2, a_src2, a_dst2, b2, lw1, lb1, lw2, lb2):
    src_r = edge_index[0].reshape(NS, NB, K)
    dst_r = edge_index[1].reshape(NS, NB, K)
    h1, atab1 = _tc_mm([(x, W1)], _pack_a(a_src1, a_dst1))
    g1 = _sc_gat_layer(h1, atab1.T.reshape(H, 2 * N), src_r, dst_r,
                       b1.reshape(H, C))
    h2, atab2 = _tc_mm([(g1[0:N], W2[0:C]), (g1[N:2 * N], W2[C:2 * C])],
                       _pack_a(a_src2, a_dst2))
    g2 = _sc_gat_layer(h2, atab2.T.reshape(H, 2 * N), src_r, dst_r,
                       b2.reshape(H, C))
    return _tc_post(g2, lw1, lb1, lw2, lb2)
